# baseline (device time: 9806 ns/iter reference)
import jax
import jax.numpy as jnp
from jax import lax
from jax.experimental import pallas as pl
from jax.experimental.pallas import tpu as pltpu

C = 4


def kernel(x):
    _, m, n = x.shape
    half = n // 2
    rows = m // C

    def body(
        x_hbm,
        out_ref,
        fstage,
        flocal,
        qstage,
        qrecv,
        sscale,
        rscale,
        in_sems,
        loc_sem,
        qsend_sems,
        qrecv_sems,
        ssend_sems,
        srecv_sems,
    ):
        my_x = lax.axis_index("x")
        my_y = lax.axis_index("y")
        other_y = 1 - my_y
        col0 = my_y * half
        scol0 = other_y * half

        def in_dma(c):
            return pltpu.make_async_copy(
                x_hbm.at[0, pl.ds(c * rows, rows), pl.ds(scol0, half)],
                fstage.at[c],
                in_sems.at[c],
            )

        local_dma = pltpu.make_async_copy(
            x_hbm.at[0, :, pl.ds(col0, half)], flocal, loc_sem
        )

        def scale_rdma(c):
            return pltpu.make_async_remote_copy(
                src_ref=sscale.at[c],
                dst_ref=rscale.at[c],
                send_sem=ssend_sems.at[c],
                recv_sem=srecv_sems.at[c],
                device_id=(my_x, other_y),
                device_id_type=pl.DeviceIdType.MESH,
            )

        def chunk_rdma(c):
            return pltpu.make_async_remote_copy(
                src_ref=qstage.at[c],
                dst_ref=qrecv.at[c],
                send_sem=qsend_sems.at[c],
                recv_sem=qrecv_sems.at[c],
                device_id=(my_x, other_y),
                device_id_type=pl.DeviceIdType.MESH,
            )

        barrier_sem = pltpu.get_barrier_semaphore()
        pl.semaphore_signal(
            barrier_sem,
            inc=1,
            device_id=(my_x, other_y),
            device_id_type=pl.DeviceIdType.MESH,
        )
        for c in range(C):
            in_dma(c).start()
        local_dma.start()
        pl.semaphore_wait(barrier_sem, 1)

        for c in range(C):
            in_dma(c).wait()
            m_abs = jnp.max(jnp.abs(fstage[c])) + 1e-30
            sscale[c] = jnp.full((8, 128), m_abs / 127.0, jnp.float32)
            qstage[c] = jnp.clip(
                jnp.round(fstage[c] * (127.0 / m_abs)), -127.0, 127.0
            ).astype(jnp.int8)
            scale_rdma(c).start()
            chunk_rdma(c).start()

        local_dma.wait()
        for c in range(C):
            scale_rdma(c).wait_recv()
            chunk_rdma(c).wait_recv()
            rs = jnp.max(rscale[c])
            out_ref[pl.ds(c * rows, rows), :] = (
                flocal[pl.ds(c * rows, rows), :]
                + qrecv[c].astype(jnp.float32) * rs
            ).astype(jnp.bfloat16)

        for c in range(C):
            scale_rdma(c).wait_send()
            chunk_rdma(c).wait_send()

    return pl.pallas_call(
        body,
        out_shape=jax.ShapeDtypeStruct((m, half), jnp.bfloat16),
        in_specs=[pl.BlockSpec(memory_space=pl.ANY)],
        out_specs=pl.BlockSpec(memory_space=pltpu.VMEM),
        scratch_shapes=[
            pltpu.VMEM((C, rows, half), jnp.float32),
            pltpu.VMEM((m, half), jnp.float32),
            pltpu.VMEM((C, rows, half), jnp.int8),
            pltpu.VMEM((C, rows, half), jnp.int8),
            pltpu.VMEM((C, 8, 128), jnp.float32),
            pltpu.VMEM((C, 8, 128), jnp.float32),
            pltpu.SemaphoreType.DMA((C,)),
            pltpu.SemaphoreType.DMA,
            pltpu.SemaphoreType.DMA((C,)),
            pltpu.SemaphoreType.DMA((C,)),
            pltpu.SemaphoreType.DMA((C,)),
            pltpu.SemaphoreType.DMA((C,)),
        ],
        compiler_params=pltpu.CompilerParams(collective_id=0),
    )(x)


# device time: 9723 ns/iter; 1.0085x vs baseline; 1.0085x over previous
import jax
import jax.numpy as jnp
from jax import lax
from jax.experimental import pallas as pl
from jax.experimental.pallas import tpu as pltpu

C = 4


def kernel(x):
    _, m, n = x.shape
    half = n // 2
    rows = m // C

    def body(
        x_ref,
        out_hbm,
        qstage,
        qrecv,
        sscale,
        rscale,
        obuf,
        qsend_sems,
        qrecv_sems,
        ssend_sems,
        srecv_sems,
        out_sems,
    ):
        my_x = lax.axis_index("x")
        my_y = lax.axis_index("y")
        other_y = 1 - my_y
        col0 = my_y * half
        scol0 = other_y * half

        barrier_sem = pltpu.get_barrier_semaphore()
        pl.semaphore_signal(
            barrier_sem,
            inc=1,
            device_id=(my_x, other_y),
            device_id_type=pl.DeviceIdType.MESH,
        )

        def scale_rdma(c):
            return pltpu.make_async_remote_copy(
                src_ref=sscale.at[c],
                dst_ref=rscale.at[c],
                send_sem=ssend_sems.at[c],
                recv_sem=srecv_sems.at[c],
                device_id=(my_x, other_y),
                device_id_type=pl.DeviceIdType.MESH,
            )

        def chunk_rdma(c):
            return pltpu.make_async_remote_copy(
                src_ref=qstage.at[c],
                dst_ref=qrecv.at[c],
                send_sem=qsend_sems.at[c],
                recv_sem=qrecv_sems.at[c],
                device_id=(my_x, other_y),
                device_id_type=pl.DeviceIdType.MESH,
            )

        def out_dma(c):
            return pltpu.make_async_copy(
                obuf.at[c],
                out_hbm.at[pl.ds(c * rows, rows), :],
                out_sems.at[c],
            )

        for c in range(C):
            v = x_ref[0, pl.ds(c * rows, rows), pl.ds(scol0, half)]
            m_abs = jnp.max(jnp.abs(v)) + 1e-30
            sscale[c] = jnp.full((8, 128), m_abs / 127.0, jnp.float32)
            qstage[c] = jnp.clip(
                jnp.round(v * (127.0 / m_abs)), -127.0, 127.0
            ).astype(jnp.int8)
            if c == 0:
                pl.semaphore_wait(barrier_sem, 1)
            scale_rdma(c).start()
            chunk_rdma(c).start()

        for c in range(C):
            scale_rdma(c).wait_recv()
            chunk_rdma(c).wait_recv()
            rs = jnp.max(rscale[c])
            obuf[c] = (
                x_ref[0, pl.ds(c * rows, rows), pl.ds(col0, half)]
                + qrecv[c].astype(jnp.float32) * rs
            ).astype(jnp.bfloat16)
            out_dma(c).start()

        for c in range(C):
            out_dma(c).wait()
            scale_rdma(c).wait_send()
            chunk_rdma(c).wait_send()

    return pl.pallas_call(
        body,
        out_shape=jax.ShapeDtypeStruct((m, half), jnp.bfloat16),
        in_specs=[pl.BlockSpec(memory_space=pltpu.VMEM)],
        out_specs=pl.BlockSpec(memory_space=pl.ANY),
        scratch_shapes=[
            pltpu.VMEM((C, rows, half), jnp.int8),
            pltpu.VMEM((C, rows, half), jnp.int8),
            pltpu.VMEM((C, 8, 128), jnp.float32),
            pltpu.VMEM((C, 8, 128), jnp.float32),
            pltpu.VMEM((C, rows, half), jnp.bfloat16),
            pltpu.SemaphoreType.DMA((C,)),
            pltpu.SemaphoreType.DMA((C,)),
            pltpu.SemaphoreType.DMA((C,)),
            pltpu.SemaphoreType.DMA((C,)),
            pltpu.SemaphoreType.DMA((C,)),
        ],
        compiler_params=pltpu.CompilerParams(collective_id=0),
    )(x)
